# Initial kernel scaffold; baseline (speedup 1.0000x reference)
import jax
import jax.numpy as jnp
from jax.experimental import pallas as pl


def kernel(x, weight, bias):
    raise NotImplementedError("write your pallas kernel here")


# speedup vs baseline: 1.6327x; 1.6327x over previous
import jax
import jax.numpy as jnp
from jax.experimental import pallas as pl
from jax.experimental.pallas import tpu as pltpu

_BM = 1024
_BN = 1024
_VMEM_LIMIT = 60 * 1024 * 1024


def _matmul_bias_kernel(x_ref, w_ref, b_ref, o_ref):
    o_ref[...] = (
        jax.lax.dot_general(
            x_ref[...],
            w_ref[...],
            dimension_numbers=(((1,), (1,)), ((), ())),
            preferred_element_type=jnp.float32,
        )
        + b_ref[...]
    )


@jax.jit
def kernel(x, weight, bias):
    B, D_in = x.shape
    D_out = weight.shape[0]

    x_bf = x.astype(jnp.bfloat16)
    w_bf = weight.astype(jnp.bfloat16)
    b2 = bias.reshape(1, D_out)

    m_grid = B // _BM
    n_grid = D_out // _BN

    return pl.pallas_call(
        _matmul_bias_kernel,
        out_shape=jax.ShapeDtypeStruct((B, D_out), jnp.float32),
        grid=(m_grid, n_grid),
        in_specs=[
            pl.BlockSpec((_BM, D_in), lambda i, j: (i, 0)),
            pl.BlockSpec((_BN, D_in), lambda i, j: (j, 0)),
            pl.BlockSpec((1, _BN), lambda i, j: (0, j)),
        ],
        out_specs=pl.BlockSpec((_BM, _BN), lambda i, j: (i, j)),
        compiler_params=pltpu.CompilerParams(
            dimension_semantics=("parallel", "parallel"),
            vmem_limit_bytes=_VMEM_LIMIT,
        ),
    )(x_bf, w_bf, b2)


# speedup vs baseline: 1.8785x; 1.1505x over previous
import jax
import jax.numpy as jnp
from jax.experimental import pallas as pl
from jax.experimental.pallas import tpu as pltpu

_BM = 1024
_BN = 1024
_VMEM_LIMIT = 64 * 1024 * 1024


def _matmul_bias_kernel(x_ref, w_ref, b_ref, o_ref):
    o_ref[...] = (
        jax.lax.dot_general(
            x_ref[...].astype(jnp.bfloat16),
            w_ref[...],
            dimension_numbers=(((1,), (1,)), ((), ())),
            preferred_element_type=jnp.float32,
        )
        + b_ref[...]
    )


@jax.jit
def kernel(x, weight, bias):
    B, D_in = x.shape
    D_out = weight.shape[0]

    w_bf = weight.astype(jnp.bfloat16)
    b2 = bias.reshape(1, D_out)

    m_grid = B // _BM
    n_grid = D_out // _BN

    return pl.pallas_call(
        _matmul_bias_kernel,
        out_shape=jax.ShapeDtypeStruct((B, D_out), jnp.float32),
        grid=(m_grid, n_grid),
        in_specs=[
            pl.BlockSpec((_BM, D_in), lambda i, j: (i, 0)),
            pl.BlockSpec((_BN, D_in), lambda i, j: (j, 0)),
            pl.BlockSpec((1, _BN), lambda i, j: (0, j)),
        ],
        out_specs=pl.BlockSpec((_BM, _BN), lambda i, j: (i, j)),
        compiler_params=pltpu.CompilerParams(
            dimension_semantics=("parallel", "parallel"),
            vmem_limit_bytes=_VMEM_LIMIT,
        ),
    )(x, w_bf, b2)
